# baseline (speedup 1.0000x reference)
import functools

import jax
import jax.numpy as jnp
from jax import lax
from jax.experimental import pallas as pl
from jax.experimental.pallas import tpu as pltpu
from jax.experimental.pallas import tpu_sc as plsc

DIM = 128
BATCH = 4096

NUM_CORES = 2
NUM_SUBCORES = 16
NW = NUM_CORES * NUM_SUBCORES
CHUNK = BATCH // NW
HALF = CHUNK // 2


def _body(center_hbm, id2center_hbm, doc_hbm, neg_hbm, dc_hbm, nc_hbm,
          idx_v, cid_v, rows_v, *sems):
    wid = lax.axis_index("s") * NUM_CORES + lax.axis_index("c")
    base = wid * CHUNK
    ids_refs = (doc_hbm, neg_hbm)
    out_refs = (dc_hbm, nc_hbm)
    s_stage, s_cid = sems[0:2], sems[2:4]
    s_rows, s_out = sems[4:8], sems[8:12]

    stage = [
        pltpu.async_copy(ids_refs[j].at[pl.ds(base, CHUNK)], idx_v.at[j],
                         s_stage[j])
        for j in range(2)
    ]
    cid = []
    for j in range(2):
        stage[j].wait()
        cid.append(pltpu.async_copy(id2center_hbm.at[idx_v.at[j]],
                                    cid_v.at[j], s_cid[j]))
    rows = []
    for j in range(2):
        cid[j].wait()
        for h in range(2):
            rows.append(pltpu.async_copy(
                center_hbm.at[cid_v.at[j, pl.ds(h * HALF, HALF)]],
                rows_v.at[j, h], s_rows[2 * j + h]))
    outs = []
    for j in range(2):
        for h in range(2):
            rows[2 * j + h].wait()
            outs.append(pltpu.async_copy(
                rows_v.at[j, h],
                out_refs[j].at[pl.ds(base + h * HALF, HALF)],
                s_out[2 * j + h]))
    for k in range(4):
        outs[k].wait()


@jax.jit
def _ivf_lookup(center_vecs, id2center, doc_ids, neg_ids):
    run = functools.partial(
        pl.kernel,
        out_type=(
            jax.ShapeDtypeStruct((BATCH, DIM), jnp.float32),
            jax.ShapeDtypeStruct((BATCH, DIM), jnp.float32),
        ),
        mesh=plsc.VectorSubcoreMesh(core_axis_name="c", subcore_axis_name="s"),
        scratch_types=[
            pltpu.VMEM((2, CHUNK), jnp.int32),
            pltpu.VMEM((2, CHUNK), jnp.int32),
            pltpu.VMEM((2, 2, HALF, DIM), jnp.float32),
        ] + [pltpu.SemaphoreType.DMA] * 12,
    )(_body)
    return run(center_vecs, id2center, doc_ids, neg_ids)


def kernel(center_vecs, id2center, doc_ids, neg_ids):
    return _ivf_lookup(center_vecs, id2center, doc_ids, neg_ids)
